# initial kernel scaffold (unmeasured)
import jax
import jax.numpy as jnp
from jax import lax
from jax.experimental import pallas as pl
from jax.experimental.pallas import tpu as pltpu

D_OUT = 1024
F = 4096
HALF = D_OUT // 2


def kernel(x, dy):
    m_per, d = x.shape
    _, f = dy.shape
    assert d == D_OUT and f == F

    def body(x_ref, dy_ref, out_ref, send_buf, recv_buf, send_sem, recv_sem):
        mx = lax.axis_index("x")
        my = lax.axis_index("y")
        mz = lax.axis_index("z")
        px = 1 - mx

        send_buf[...] = lax.dot_general(
            x_ref[:, pl.ds(px * HALF, HALF)],
            dy_ref[...],
            dimension_numbers=(((0,), (0,)), ((), ())),
            preferred_element_type=jnp.float32,
        )

        barrier_sem = pltpu.get_barrier_semaphore()
        pl.semaphore_signal(
            barrier_sem, inc=1,
            device_id=(px, my, mz), device_id_type=pl.DeviceIdType.MESH,
        )
        pl.semaphore_wait(barrier_sem, 1)

        rdma = pltpu.make_async_remote_copy(
            src_ref=send_buf,
            dst_ref=recv_buf,
            send_sem=send_sem,
            recv_sem=recv_sem,
            device_id=(px, my, mz),
            device_id_type=pl.DeviceIdType.MESH,
        )
        rdma.start()

        mine = lax.dot_general(
            x_ref[:, pl.ds(mx * HALF, HALF)],
            dy_ref[...],
            dimension_numbers=(((0,), (0,)), ((), ())),
            preferred_element_type=jnp.float32,
        )

        rdma.wait()
        out_ref[...] = mine + recv_buf[...]

    return pl.pallas_call(
        body,
        out_shape=jax.ShapeDtypeStruct((HALF, F), jnp.float32),
        in_specs=[
            pl.BlockSpec(memory_space=pltpu.VMEM),
            pl.BlockSpec(memory_space=pltpu.VMEM),
        ],
        out_specs=pl.BlockSpec(memory_space=pltpu.VMEM),
        scratch_shapes=[
            pltpu.VMEM((HALF, F), jnp.float32),
            pltpu.VMEM((HALF, F), jnp.float32),
            pltpu.SemaphoreType.DMA,
            pltpu.SemaphoreType.DMA,
        ],
        compiler_params=pltpu.CompilerParams(collective_id=0),
    )(x, dy)


# baseline (device time: 115518 ns/iter reference)
import jax
import jax.numpy as jnp
from jax import lax
from jax.experimental import pallas as pl
from jax.experimental.pallas import tpu as pltpu

D_OUT = 1024
F = 4096
HALF = D_OUT // 2


def kernel(x, dy):
    m_per, d = x.shape
    _, f = dy.shape
    assert d == D_OUT and f == F

    def body(x_ref, dy_ref, out_ref, send_buf, recv_buf, send_sem, recv_sem):
        mx = lax.axis_index("x")
        my = lax.axis_index("y")
        mz = lax.axis_index("z")
        px = 1 - mx

        send_buf[...] = lax.dot_general(
            x_ref[:, pl.ds(px * HALF, HALF)],
            dy_ref[...],
            dimension_numbers=(((0,), (0,)), ((), ())),
            preferred_element_type=jnp.float32,
        )

        barrier_sem = pltpu.get_barrier_semaphore()
        pl.semaphore_signal(
            barrier_sem, inc=1,
            device_id=(px, my, mz), device_id_type=pl.DeviceIdType.MESH,
        )
        pl.semaphore_wait(barrier_sem, 1)

        rdma = pltpu.make_async_remote_copy(
            src_ref=send_buf,
            dst_ref=recv_buf,
            send_sem=send_sem,
            recv_sem=recv_sem,
            device_id=(px, my, mz),
            device_id_type=pl.DeviceIdType.MESH,
        )
        rdma.start()

        out_ref[...] = lax.dot_general(
            x_ref[:, pl.ds(mx * HALF, HALF)],
            dy_ref[...],
            dimension_numbers=(((0,), (0,)), ((), ())),
            preferred_element_type=jnp.float32,
        )

        rdma.wait()
        out_ref[...] += recv_buf[...]

    return pl.pallas_call(
        body,
        out_shape=jax.ShapeDtypeStruct((HALF, F), jnp.float32),
        in_specs=[
            pl.BlockSpec(memory_space=pltpu.VMEM),
            pl.BlockSpec(memory_space=pltpu.VMEM),
        ],
        out_specs=pl.BlockSpec(memory_space=pltpu.VMEM),
        scratch_shapes=[
            pltpu.VMEM((HALF, F), jnp.float32),
            pltpu.VMEM((HALF, F), jnp.float32),
            pltpu.SemaphoreType.DMA,
            pltpu.SemaphoreType.DMA,
        ],
        compiler_params=pltpu.CompilerParams(
            collective_id=0,
            vmem_limit_bytes=100 * 1024 * 1024,
        ),
    )(x, dy)


# device time: 74984 ns/iter; 1.5406x vs baseline; 1.5406x over previous
import jax
import jax.numpy as jnp
from jax import lax
from jax.experimental import pallas as pl
from jax.experimental.pallas import tpu as pltpu

D_OUT = 1024
F = 4096
HALF = D_OUT // 2
Q = HALF // 2
NC = 8
FC = F // NC

_CONTRACT0 = (((0,), (0,)), ((), ()))


def kernel(x, dy):
    m_per, d = x.shape
    _, f = dy.shape
    assert d == D_OUT and f == F

    def body(x_ref, dy_ref, out_ref, sendx, xq, yq,
             xs_sems, xr_sems, ys_sems, yr_sems):
        mx = lax.axis_index("x")
        my = lax.axis_index("y")
        mz = lax.axis_index("z")
        px = 1 - mx
        py = 1 - my
        xpeer = (px, my, mz)
        ypeer = (mx, py, mz)

        barrier_sem = pltpu.get_barrier_semaphore()
        for nbr in (xpeer, ypeer):
            pl.semaphore_signal(
                barrier_sem, inc=1,
                device_id=nbr, device_id_type=pl.DeviceIdType.MESH,
            )
        pl.semaphore_wait(barrier_sem, 2)

        qsend_off = px * HALF + my * Q
        x_rdmas = []
        for c in range(NC):
            sendx[c, :, :] = lax.dot_general(
                x_ref[:, pl.ds(qsend_off, Q)],
                dy_ref[:, pl.ds(c * FC, FC)],
                dimension_numbers=_CONTRACT0,
                preferred_element_type=jnp.float32,
            )
            r = pltpu.make_async_remote_copy(
                src_ref=sendx.at[c],
                dst_ref=xq.at[c],
                send_sem=xs_sems.at[c],
                recv_sem=xr_sems.at[c],
                device_id=xpeer,
                device_id_type=pl.DeviceIdType.MESH,
            )
            r.start()
            x_rdmas.append(r)

        out_ref[...] = lax.dot_general(
            x_ref[:, pl.ds(mx * HALF, HALF)],
            dy_ref[...],
            dimension_numbers=_CONTRACT0,
            preferred_element_type=jnp.float32,
        )

        q0 = my * Q
        q1 = py * Q
        y_rdmas = []
        for c in range(NC):
            x_rdmas[c].wait_recv()
            r = pltpu.make_async_remote_copy(
                src_ref=xq.at[c],
                dst_ref=yq.at[c],
                send_sem=ys_sems.at[c],
                recv_sem=yr_sems.at[c],
                device_id=ypeer,
                device_id_type=pl.DeviceIdType.MESH,
            )
            r.start()
            y_rdmas.append(r)
            out_ref[pl.ds(q0, Q), pl.ds(c * FC, FC)] += xq[c, :, :]

        for c in range(NC):
            y_rdmas[c].wait_recv()
            out_ref[pl.ds(q1, Q), pl.ds(c * FC, FC)] += yq[c, :, :]

        for c in range(NC):
            x_rdmas[c].wait_send()
            y_rdmas[c].wait_send()

    return pl.pallas_call(
        body,
        out_shape=jax.ShapeDtypeStruct((HALF, F), jnp.float32),
        in_specs=[
            pl.BlockSpec(memory_space=pltpu.VMEM),
            pl.BlockSpec(memory_space=pltpu.VMEM),
        ],
        out_specs=pl.BlockSpec(memory_space=pltpu.VMEM),
        scratch_shapes=[
            pltpu.VMEM((NC, Q, FC), jnp.float32),
            pltpu.VMEM((NC, Q, FC), jnp.float32),
            pltpu.VMEM((NC, Q, FC), jnp.float32),
            pltpu.SemaphoreType.DMA((NC,)),
            pltpu.SemaphoreType.DMA((NC,)),
            pltpu.SemaphoreType.DMA((NC,)),
            pltpu.SemaphoreType.DMA((NC,)),
        ],
        compiler_params=pltpu.CompilerParams(
            collective_id=0,
            vmem_limit_bytes=100 * 1024 * 1024,
        ),
    )(x, dy)


# device time: 73657 ns/iter; 1.5683x vs baseline; 1.0180x over previous
import jax
import jax.numpy as jnp
from jax import lax
from jax.experimental import pallas as pl
from jax.experimental.pallas import tpu as pltpu

D_OUT = 1024
F = 4096
HALF = D_OUT // 2
Q = HALF // 2
NC = 8
FC = F // NC

_CONTRACT0 = (((0,), (0,)), ((), ()))


def kernel(x, dy):
    m_per, d = x.shape
    _, f = dy.shape
    assert d == D_OUT and f == F

    def body(x_ref, dy_ref, out_ref, sendx, xq, yq,
             xs_sems, xr_sems, ys_sems, yr_sems):
        mx = lax.axis_index("x")
        my = lax.axis_index("y")
        mz = lax.axis_index("z")
        px = 1 - mx
        py = 1 - my
        xpeer = (px, my, mz)
        ypeer = (mx, py, mz)

        barrier_sem = pltpu.get_barrier_semaphore()
        for nbr in (xpeer, ypeer):
            pl.semaphore_signal(
                barrier_sem, inc=1,
                device_id=nbr, device_id_type=pl.DeviceIdType.MESH,
            )
        pl.semaphore_wait(barrier_sem, 2)

        qsend_off = px * HALF + my * Q
        x_rdmas = []
        for c in range(NC):
            sendx[c, :, :] = lax.dot_general(
                x_ref[:, pl.ds(qsend_off, Q)],
                dy_ref[:, pl.ds(c * FC, FC)],
                dimension_numbers=_CONTRACT0,
                preferred_element_type=jnp.float32,
            )
            r = pltpu.make_async_remote_copy(
                src_ref=sendx.at[c],
                dst_ref=xq.at[c],
                send_sem=xs_sems.at[c],
                recv_sem=xr_sems.at[c],
                device_id=xpeer,
                device_id_type=pl.DeviceIdType.MESH,
            )
            r.start()
            x_rdmas.append(r)

        q0 = my * Q
        q1 = py * Q
        y_rdmas = []
        for c in range(NC):
            x_rdmas[c].wait_recv()
            r = pltpu.make_async_remote_copy(
                src_ref=xq.at[c],
                dst_ref=yq.at[c],
                send_sem=ys_sems.at[c],
                recv_sem=yr_sems.at[c],
                device_id=ypeer,
                device_id_type=pl.DeviceIdType.MESH,
            )
            r.start()
            y_rdmas.append(r)
            out_ref[:, pl.ds(c * FC, FC)] = lax.dot_general(
                x_ref[:, pl.ds(mx * HALF, HALF)],
                dy_ref[:, pl.ds(c * FC, FC)],
                dimension_numbers=_CONTRACT0,
                preferred_element_type=jnp.float32,
            )
            out_ref[pl.ds(q0, Q), pl.ds(c * FC, FC)] += xq[c, :, :]

        for c in range(NC):
            y_rdmas[c].wait_recv()
            out_ref[pl.ds(q1, Q), pl.ds(c * FC, FC)] += yq[c, :, :]

        for c in range(NC):
            x_rdmas[c].wait_send()
            y_rdmas[c].wait_send()

    return pl.pallas_call(
        body,
        out_shape=jax.ShapeDtypeStruct((HALF, F), jnp.float32),
        in_specs=[
            pl.BlockSpec(memory_space=pltpu.VMEM),
            pl.BlockSpec(memory_space=pltpu.VMEM),
        ],
        out_specs=pl.BlockSpec(memory_space=pltpu.VMEM),
        scratch_shapes=[
            pltpu.VMEM((NC, Q, FC), jnp.float32),
            pltpu.VMEM((NC, Q, FC), jnp.float32),
            pltpu.VMEM((NC, Q, FC), jnp.float32),
            pltpu.SemaphoreType.DMA((NC,)),
            pltpu.SemaphoreType.DMA((NC,)),
            pltpu.SemaphoreType.DMA((NC,)),
            pltpu.SemaphoreType.DMA((NC,)),
        ],
        compiler_params=pltpu.CompilerParams(
            collective_id=0,
            vmem_limit_bytes=100 * 1024 * 1024,
        ),
    )(x, dy)
